# Initial kernel scaffold; baseline (speedup 1.0000x reference)
import jax
import jax.numpy as jnp
from jax.experimental import pallas as pl


def kernel(input, quantized_weight, weight_scale):
    raise NotImplementedError("write your pallas kernel here")


# speedup vs baseline: 1.3798x; 1.3798x over previous
import functools

import numpy as np
import jax
import jax.numpy as jnp
from jax import lax
from jax.experimental import pallas as pl
from jax.experimental.pallas import tpu as pltpu
from jax.experimental.pallas import tpu_sc as plsc

_D = 128
_NC, _NS, _L = 2, 16, 16
_NW = _NC * _NS
_CHUNK_IR = 2
_CHUNK = _CHUNK_IR * 128


@functools.lru_cache(maxsize=None)
def _make_kernel(B, V):
    assert B % (_NW * _CHUNK) == 0, B
    rows_per_w = B // _NW
    ir_per_w = rows_per_w // 128
    n_chunks = rows_per_w // _CHUNK

    mesh = plsc.VectorSubcoreMesh(core_axis_name="c", subcore_axis_name="s")

    @functools.partial(
        pl.kernel,
        mesh=mesh,
        compiler_params=pltpu.CompilerParams(
            needs_layout_passes=False, use_tc_tiling_on_sc=False),
        out_type=jax.ShapeDtypeStruct((B * _D,), jnp.float32),
        scratch_types=[
            pltpu.VMEM((_CHUNK_IR, 128), jnp.int32),
            pltpu.VMEM((_CHUNK, _D // 4), jnp.int32),
            pltpu.VMEM((_CHUNK * _D,), jnp.float32),
            pltpu.VMEM((_L,), jnp.float32),
            pltpu.SemaphoreType.DMA,
        ],
    )
    def k(idx_hbm, tbl_hbm, scale_hbm, out_hbm, idx_v, rows_v, out_v,
          scale_v, sem):
        wid = lax.axis_index("s") * _NC + lax.axis_index("c")
        pltpu.sync_copy(scale_hbm, scale_v)
        s = scale_v[...]
        lane4 = 4 * lax.iota(jnp.int32, _L)
        cols = [lane4 + (64 * h + kk) for h in range(2) for kk in range(4)]

        def do_chunk(i, carry):
            ir0 = wid * ir_per_w + i * _CHUNK_IR
            pltpu.sync_copy(idx_hbm.at[pl.ds(ir0, _CHUNK_IR)], idx_v)
            cps = [
                pltpu.async_copy(
                    tbl_hbm.at[idx_v.at[j]],
                    rows_v.at[pl.ds(j * 128, 128)],
                    sem,
                )
                for j in range(_CHUNK_IR)
            ]
            for cp in cps:
                cp.wait()

            def do_row(r, c2):
                rb = r * _D
                for h in range(2):
                    w = rows_v[r, pl.ds(16 * h, 16)]
                    for kk in range(4):
                        t = (w << (24 - 8 * kk)) >> 24
                        f = t.astype(jnp.float32) * s
                        plsc.store_scatter(out_v, [rb + cols[4 * h + kk]], f)
                return c2

            lax.fori_loop(0, _CHUNK, do_row, 0)
            out0 = (wid * rows_per_w + i * _CHUNK) * _D
            pltpu.sync_copy(out_v, out_hbm.at[pl.ds(out0, _CHUNK * _D)])
            return carry

        lax.fori_loop(0, n_chunks, do_chunk, 0)

    return k


def kernel(input, quantized_weight, weight_scale):
    B = input.size
    idx2 = input.reshape(B // 128, 128).astype(jnp.int32)
    tbl32 = jax.lax.bitcast_convert_type(
        quantized_weight.reshape(quantized_weight.shape[0], _D // 4, 4),
        jnp.int32)
    scale = jnp.broadcast_to(
        jnp.asarray(weight_scale, jnp.float32).reshape(1), (_L,))
    out = _make_kernel(B, quantized_weight.shape[0])(idx2, tbl32, scale)
    return out.reshape(input.shape + (_D,))


# speedup vs baseline: 2.0966x; 1.5195x over previous
import functools

import jax
import jax.numpy as jnp
from jax import lax
from jax.experimental import pallas as pl
from jax.experimental.pallas import tpu as pltpu
from jax.experimental.pallas import tpu_sc as plsc

_D = 128
_NC, _NS, _L = 2, 16, 16
_NW = _NC * _NS
_CHUNK_IR = 2
_CHUNK = _CHUNK_IR * 128


@functools.lru_cache(maxsize=None)
def _make_kernel(B, V):
    assert B % (_NW * _CHUNK) == 0, B
    rows_per_w = B // _NW
    ir_per_w = rows_per_w // 128
    n_chunks = rows_per_w // _CHUNK

    mesh = plsc.VectorSubcoreMesh(core_axis_name="c", subcore_axis_name="s")

    @functools.partial(
        pl.kernel,
        mesh=mesh,
        compiler_params=pltpu.CompilerParams(
            needs_layout_passes=False, use_tc_tiling_on_sc=False),
        out_type=jax.ShapeDtypeStruct((B, _D), jnp.float32),
        scratch_types=[
            pltpu.VMEM((_CHUNK_IR, 128), jnp.int32),
            pltpu.VMEM((_CHUNK, _D), jnp.int8),
            pltpu.VMEM((_CHUNK, _D), jnp.float32),
            pltpu.VMEM((_L,), jnp.float32),
            pltpu.SemaphoreType.DMA,
        ],
    )
    def k(idx_hbm, tbl_hbm, scale_hbm, out_hbm, idx_v, rows_v, out_v,
          scale_v, sem):
        wid = lax.axis_index("s") * _NC + lax.axis_index("c")
        pltpu.sync_copy(scale_hbm, scale_v)
        s = scale_v[...]
        lane4 = 4 * lax.iota(jnp.int32, _L)
        cols = [lane4 + (64 * h + kk) for h in range(2) for kk in range(4)]

        def do_chunk(i, carry):
            ir0 = wid * ir_per_w + i * _CHUNK_IR
            pltpu.sync_copy(idx_hbm.at[pl.ds(ir0, _CHUNK_IR)], idx_v)
            cps = [
                pltpu.async_copy(
                    tbl_hbm.at[idx_v.at[j]],
                    rows_v.at[pl.ds(j * 128, 128)],
                    sem,
                )
                for j in range(_CHUNK_IR)
            ]
            for cp in cps:
                cp.wait()

            def do_row(r, c2):
                rsplat = jnp.full((_L,), r, jnp.int32)
                for h in range(2):
                    w8 = rows_v[r, pl.ds(64 * h, 64)]
                    w = plsc.bitcast(w8, jnp.int32)
                    for kk in range(4):
                        t = (w << (24 - 8 * kk)) >> 24
                        f = t.astype(jnp.float32) * s
                        plsc.store_scatter(
                            out_v, [rsplat, cols[4 * h + kk]], f)
                return c2

            lax.fori_loop(0, _CHUNK, do_row, 0)
            row0 = wid * rows_per_w + i * _CHUNK
            pltpu.sync_copy(out_v, out_hbm.at[pl.ds(row0, _CHUNK)])
            return carry

        lax.fori_loop(0, n_chunks, do_chunk, 0)

    return k


def kernel(input, quantized_weight, weight_scale):
    B = input.size
    idx2 = input.reshape(B // 128, 128).astype(jnp.int32)
    scale = jnp.broadcast_to(
        jnp.asarray(weight_scale, jnp.float32).reshape(1), (_L,))
    out = _make_kernel(B, quantized_weight.shape[0])(
        idx2, quantized_weight, scale)
    return out.reshape(input.shape + (_D,))
